# baseline (speedup 1.0000x reference)
import functools

import jax
import jax.numpy as jnp
import numpy as np
from jax.experimental import pallas as pl
from jax.experimental.pallas import tpu as pltpu

B = 1
SEQ = 2048
DIM = 1024
DEPTH = 2
HEADS = 16
DH = 64
BUCKET = 64
NH = 4
OUT = 2
MAXSEQ = 2048
NB = SEQ // BUCKET
NCH = NH * NB

_ROT = jnp.asarray(
    np.random.default_rng(12345).standard_normal((DEPTH, DH, NH, NB // 2)),
    dtype=jnp.float32,
)


def _mm_body(x_ref, w_ref, b_ref, g_ref, bb_ref, r_ref, o_ref, *, ln, act, res):
    x = x_ref[...]
    if ln:
        m = jnp.mean(x, axis=-1, keepdims=True)
        v = jnp.mean((x - m) ** 2, axis=-1, keepdims=True)
        x = (x - m) * jax.lax.rsqrt(v + 1e-5) * g_ref[...] + bb_ref[...]
    acc = jnp.dot(x, w_ref[...], preferred_element_type=jnp.float32)
    acc = acc + b_ref[...]
    if act == "gelu":
        acc = 0.5 * acc * (1.0 + jax.lax.erf(acc * (2.0 ** -0.5)))
    if res:
        acc = acc + r_ref[...]
    o_ref[...] = acc


def _ln_mm(x, w, bias, gamma=None, beta=None, resid=None, act=None,
           bm=256, bn=512):
    M, K = x.shape
    N = w.shape[1]
    ln = gamma is not None
    res = resid is not None
    if not ln:
        gamma = jnp.zeros((1, 1), jnp.float32)
        beta = jnp.zeros((1, 1), jnp.float32)
    else:
        gamma = gamma.reshape(1, K)
        beta = beta.reshape(1, K)
    if not res:
        resid = jnp.zeros((1, 1), jnp.float32)
    bias = bias.reshape(1, N)
    gk = gamma.shape[1]
    grid = (M // bm, N // bn)
    return pl.pallas_call(
        functools.partial(_mm_body, ln=ln, act=act, res=res),
        grid=grid,
        in_specs=[
            pl.BlockSpec((bm, K), lambda i, j: (i, 0)),
            pl.BlockSpec((K, bn), lambda i, j: (0, j)),
            pl.BlockSpec((1, bn), lambda i, j: (0, j)),
            pl.BlockSpec((1, gk), lambda i, j: (0, 0)),
            pl.BlockSpec((1, gk), lambda i, j: (0, 0)),
            (pl.BlockSpec((bm, bn), lambda i, j: (i, j)) if res
             else pl.BlockSpec((1, 1), lambda i, j: (0, 0))),
        ],
        out_specs=pl.BlockSpec((bm, bn), lambda i, j: (i, j)),
        out_shape=jax.ShapeDtypeStruct((M, N), jnp.float32),
    )(x, w, bias, gamma, beta, resid)


def _head_body(x1_ref, x2_ref, g_ref, b_ref, w_ref, bo_ref, o_ref):
    x = (x1_ref[...] + x2_ref[...]) * 0.5
    m = jnp.mean(x, axis=-1, keepdims=True)
    v = jnp.mean((x - m) ** 2, axis=-1, keepdims=True)
    x = (x - m) * jax.lax.rsqrt(v + 1e-5) * g_ref[...] + b_ref[...]
    xm = jnp.mean(x, axis=0, keepdims=True)
    o_ref[...] = jnp.dot(xm, w_ref[...], preferred_element_type=jnp.float32) + bo_ref[...]


def _final_head(x1, x2, gf, bfin, wout, bout):
    return pl.pallas_call(
        _head_body,
        in_specs=[
            pl.BlockSpec((SEQ, DIM), lambda: (0, 0)),
            pl.BlockSpec((SEQ, DIM), lambda: (0, 0)),
            pl.BlockSpec((1, DIM), lambda: (0, 0)),
            pl.BlockSpec((1, DIM), lambda: (0, 0)),
            pl.BlockSpec((DIM, OUT), lambda: (0, 0)),
            pl.BlockSpec((1, OUT), lambda: (0, 0)),
        ],
        out_specs=pl.BlockSpec((1, OUT), lambda: (0, 0)),
        out_shape=jax.ShapeDtypeStruct((1, OUT), jnp.float32),
    )(x1, x2, gf.reshape(1, DIM), bfin.reshape(1, DIM), wout,
      bout.reshape(1, OUT))


def _lsh_attention(x2, g1, b1, wqk, wv, wo, bo, x1, rot):
    t = SEQ
    qkv = _ln_mm(x2, jnp.concatenate([wqk, wv], axis=1),
                 jnp.zeros((2 * HEADS * DH,), jnp.float32), g1, b1)
    qk = qkv[:, : HEADS * DH]
    v = qkv[:, HEADS * DH:]

    def split_heads(z):
        return z.reshape(t, HEADS, DH).transpose(1, 0, 2)

    qk = split_heads(qk)
    v = split_heads(v)

    rotated = jnp.einsum('atf,fni->anti', qk, rot)
    rotated = jnp.concatenate([rotated, -rotated], axis=-1)
    buckets = jnp.argmax(rotated, axis=-1)
    buckets = buckets + (jnp.arange(NH) * NB)[None, :, None]
    buckets = buckets.reshape(HEADS, NH * t)
    ticker = jnp.arange(NH * t)[None, :]
    buckets_and_t = t * buckets + (ticker % t)
    sticker = jnp.argsort(buckets_and_t, axis=-1)
    undo_sort = jnp.argsort(sticker, axis=-1)
    st = sticker % t
    sqk = jnp.take_along_axis(qk, st[..., None], axis=1)
    sv = jnp.take_along_axis(v, st[..., None], axis=1)
    bq_t = st.reshape(HEADS, NCH, BUCKET)
    bqk = sqk.reshape(HEADS, NCH, BUCKET, DH)
    bv = sv.reshape(HEADS, NCH, BUCKET, DH)
    bq = bqk
    bk = bqk / jnp.maximum(jnp.linalg.norm(bqk, axis=-1, keepdims=True), 1e-12)

    def look_one_back(z):
        zp = jnp.concatenate([z[:, -1:], z[:, :-1]], axis=1)
        return jnp.concatenate([z, zp], axis=2)

    bk = look_one_back(bk)
    bv2 = look_one_back(bv)
    bkv_t = look_one_back(bq_t)
    dots = jnp.einsum('hcie,hcje->hcij', bq, bk) * (DH ** -0.5)
    self_mask = bq_t[..., :, None] == bkv_t[..., None, :]
    dots = jnp.where(self_mask, -5e4, dots)
    dots_lse = jax.nn.logsumexp(dots, axis=-1, keepdims=True)
    p = jnp.exp(dots - dots_lse)
    bo_ = jnp.einsum('hcij,hcje->hcie', p, bv2)
    so = bo_.reshape(HEADS, -1, DH)
    slogits = dots_lse.reshape(HEADS, -1)
    o = jnp.take_along_axis(so, undo_sort[..., None], axis=1)
    logits = jnp.take_along_axis(slogits, undo_sort, axis=1)
    o = o.reshape(HEADS, NH, t, DH)
    logits = logits.reshape(HEADS, NH, t, 1)
    probs = jnp.exp(logits - jax.nn.logsumexp(logits, axis=1, keepdims=True))
    out = jnp.sum(o * probs, axis=1)
    out = out.transpose(1, 0, 2).reshape(t, HEADS * DH)
    return _ln_mm(out, wo, bo, resid=x1)


def kernel(x, params):
    p = params
    h = x[0] + p['pos_emb'][:SEQ]
    x1 = h
    x2 = h
    for i in range(DEPTH):
        y1 = _lsh_attention(x2, p['g1'][i], p['b1'][i], p['Wqk'][i],
                            p['Wv'][i], p['Wo'][i], p['bo'][i], x1, _ROT[i])
        ff = _ln_mm(y1, p['W1'][i], p['bf1'][i], p['g2'][i], p['b2'][i],
                    act="gelu")
        y2 = _ln_mm(ff, p['W2'][i], p['bf2'][i], resid=x2)
        x1, x2 = y1, y2
    return _final_head(x1, x2, p['gf'], p['bfin'], p['Wout'], p['bout'])


# speedup vs baseline: 1.8261x; 1.8261x over previous
import functools

import jax
import jax.numpy as jnp
import numpy as np
from jax import lax
from jax.experimental import pallas as pl
from jax.experimental.pallas import tpu as pltpu

B = 1
SEQ = 2048
DIM = 1024
DEPTH = 2
HEADS = 16
DH = 64
BUCKET = 64
NH = 4
OUT = 2
MAXSEQ = 2048
NB = SEQ // BUCKET
NCH = NH * NB
S = NH * SEQ
CPS = SEQ // BUCKET

_ROT = jnp.asarray(
    np.random.default_rng(12345).standard_normal((DEPTH, DH, NH, NB // 2)),
    dtype=jnp.float32,
)
_LT128 = jnp.asarray(np.tril(np.ones((128, 128), np.float32), -1))


def _mm_body(x_ref, w_ref, b_ref, g_ref, bb_ref, r_ref, o_ref, *, ln, act, res):
    x = x_ref[...]
    if ln:
        m = jnp.mean(x, axis=-1, keepdims=True)
        v = jnp.mean((x - m) ** 2, axis=-1, keepdims=True)
        x = (x - m) * jax.lax.rsqrt(v + 1e-5) * g_ref[...] + bb_ref[...]
    acc = jnp.dot(x, w_ref[...], preferred_element_type=jnp.float32)
    acc = acc + b_ref[...]
    if act == "gelu":
        acc = 0.5 * acc * (1.0 + jax.lax.erf(acc * (2.0 ** -0.5)))
    if res:
        acc = acc + r_ref[...]
    o_ref[...] = acc


def _ln_mm(x, w, bias, gamma=None, beta=None, resid=None, act=None,
           bm=256, bn=512):
    M, K = x.shape
    N = w.shape[1]
    ln = gamma is not None
    res = resid is not None
    if not ln:
        gamma = jnp.zeros((1, 1), jnp.float32)
        beta = jnp.zeros((1, 1), jnp.float32)
    else:
        gamma = gamma.reshape(1, K)
        beta = beta.reshape(1, K)
    if not res:
        resid = jnp.zeros((1, 1), jnp.float32)
    bias = bias.reshape(1, N)
    gk = gamma.shape[1]
    grid = (M // bm, N // bn)
    return pl.pallas_call(
        functools.partial(_mm_body, ln=ln, act=act, res=res),
        grid=grid,
        in_specs=[
            pl.BlockSpec((bm, K), lambda i, j: (i, 0)),
            pl.BlockSpec((K, bn), lambda i, j: (0, j)),
            pl.BlockSpec((1, bn), lambda i, j: (0, j)),
            pl.BlockSpec((1, gk), lambda i, j: (0, 0)),
            pl.BlockSpec((1, gk), lambda i, j: (0, 0)),
            (pl.BlockSpec((bm, bn), lambda i, j: (i, j)) if res
             else pl.BlockSpec((1, 1), lambda i, j: (0, 0))),
        ],
        out_specs=pl.BlockSpec((bm, bn), lambda i, j: (i, j)),
        out_shape=jax.ShapeDtypeStruct((M, N), jnp.float32),
    )(x, w, bias, gamma, beta, resid)


def _hash_body(qk_ref, rot_ref, o_ref):
    half = NB // 2
    for g in range(2):
        x = qk_ref[:, g * DH:(g + 1) * DH]
        r = jnp.dot(x, rot_ref[...], preferred_element_type=jnp.float32)
        for rr in range(NH):
            sl = r[:, rr * half:(rr + 1) * half]
            full = jnp.concatenate([sl, -sl], axis=1)
            mx = jnp.max(full, axis=1, keepdims=True)
            ii = lax.broadcasted_iota(jnp.int32, (SEQ, NB), 1)
            am = jnp.min(jnp.where(full == mx, ii, NB), axis=1, keepdims=True)
            o_ref[g, :, rr:rr + 1] = am.astype(jnp.int32)


def _lsh_hash(qkv, rotflat):
    return pl.pallas_call(
        _hash_body,
        grid=(HEADS // 2,),
        in_specs=[
            pl.BlockSpec((SEQ, 2 * DH), lambda h: (0, h)),
            pl.BlockSpec((DH, NH * NB // 2), lambda h: (0, 0)),
        ],
        out_specs=pl.BlockSpec((2, SEQ, NH), lambda h: (h, 0, 0)),
        out_shape=jax.ShapeDtypeStruct((HEADS, SEQ, NH), jnp.int32),
    )(qkv, rotflat)


def _sortperm_body(lb_ref, lt_ref, o_ref):
    lt = lt_ref[...]
    for r in range(NH):
        l = lb_ref[0, :, r:r + 1]
        oh = (l == lax.broadcasted_iota(jnp.int32, (SEQ, NB), 1))
        oh = oh.astype(jnp.float32)
        segs = []
        carry = jnp.zeros((1, NB), jnp.float32)
        for bb in range(SEQ // 128):
            blk = oh[bb * 128:(bb + 1) * 128, :]
            segs.append(jnp.dot(lt, blk, preferred_element_type=jnp.float32)
                        + carry)
            carry = carry + jnp.sum(blk, axis=0, keepdims=True)
        cum = jnp.concatenate(segs, axis=0)
        hist = carry
        ii = lax.broadcasted_iota(jnp.int32, (NB, NB), 0)
        jj = lax.broadcasted_iota(jnp.int32, (NB, NB), 1)
        m = (ii < jj).astype(jnp.float32)
        bstart = jnp.dot(hist, m, preferred_element_type=jnp.float32)
        pos = jnp.sum(oh * (cum + bstart), axis=1, keepdims=True)
        o_ref[0, :, r:r + 1] = (pos + np.float32(r * SEQ)).astype(jnp.int32)


def _sortperm(lbuck):
    return pl.pallas_call(
        _sortperm_body,
        grid=(HEADS,),
        in_specs=[
            pl.BlockSpec((1, SEQ, NH), lambda h: (h, 0, 0)),
            pl.BlockSpec((128, 128), lambda h: (0, 0)),
        ],
        out_specs=pl.BlockSpec((1, SEQ, NH), lambda h: (h, 0, 0)),
        out_shape=jax.ShapeDtypeStruct((HEADS, SEQ, NH), jnp.int32),
    )(lbuck, _LT128)


def _attn_body(sqk_ref, ptl_ref, stq_ref, stl_ref, ptt_ref, so_ref, sl_ref):
    scale = np.float32(DH ** -0.5)

    def norm(k):
        n = jnp.sqrt(jnp.sum(k * k, axis=-1, keepdims=True))
        return k / jnp.maximum(n, 1e-12)

    for c in range(CPS):
        lo = c * BUCKET
        hi = lo + BUCKET
        q = sqk_ref[0, lo:hi, :DH]
        v = sqk_ref[0, lo:hi, DH:]
        qt = stq_ref[0, lo:hi, 0:1]
        ktc = stl_ref[0, c, :, :]
        if c == 0:
            kprev = ptl_ref[0, :, :DH]
            vprev = ptl_ref[0, :, DH:]
            ktp = ptt_ref[0, 0, :, :]
        else:
            kprev = sqk_ref[0, lo - BUCKET:lo, :DH]
            vprev = sqk_ref[0, lo - BUCKET:lo, DH:]
            ktp = stl_ref[0, c - 1, :, :]
        kcat = jnp.concatenate([q, kprev], axis=0)
        vcat = jnp.concatenate([v, vprev], axis=0)
        ktrow = jnp.concatenate([ktc, ktp], axis=1)
        kn = norm(kcat)
        dots = lax.dot_general(q, kn, (((1,), (1,)), ((), ())),
                               preferred_element_type=jnp.float32) * scale
        mask = qt == ktrow
        dots = jnp.where(mask, -5e4, dots)
        mx = jnp.max(dots, axis=1, keepdims=True)
        e = jnp.exp(dots - mx)
        ssum = jnp.sum(e, axis=1, keepdims=True)
        lse = mx + jnp.log(ssum)
        p = e / ssum
        o = jnp.dot(p, vcat, preferred_element_type=jnp.float32)
        so_ref[0, lo:hi, :] = o
        sl_ref[0, lo:hi, :] = jnp.broadcast_to(lse, (BUCKET, 16))


def _chunk_attn(sqkv, stq, stl):
    return pl.pallas_call(
        _attn_body,
        grid=(HEADS, NH),
        in_specs=[
            pl.BlockSpec((1, SEQ, 2 * DH), lambda h, r: (h, r, 0)),
            pl.BlockSpec((1, BUCKET, 2 * DH),
                         lambda h, r: (h, (r * CPS - 1) % NCH, 0)),
            pl.BlockSpec((1, SEQ, 16), lambda h, r: (h, r, 0)),
            pl.BlockSpec((1, CPS, 1, BUCKET), lambda h, r: (h, r, 0, 0)),
            pl.BlockSpec((1, 1, 1, BUCKET),
                         lambda h, r: (h, (r * CPS - 1) % NCH, 0, 0)),
        ],
        out_specs=[
            pl.BlockSpec((1, SEQ, DH), lambda h, r: (h, r, 0)),
            pl.BlockSpec((1, SEQ, 16), lambda h, r: (h, r, 0)),
        ],
        out_shape=[
            jax.ShapeDtypeStruct((HEADS, S, DH), jnp.float32),
            jax.ShapeDtypeStruct((HEADS, S, 16), jnp.float32),
        ],
    )(sqkv, sqkv, stq, stl, stl)


def _comb_body(o_ref, l_ref, ctx_ref):
    o = o_ref[...]
    lg = l_ref[...][:, :, 0:1]
    m = jnp.max(lg, axis=0, keepdims=True)
    e = jnp.exp(lg - m)
    p = e / jnp.sum(e, axis=0, keepdims=True)
    ctx_ref[0] = jnp.sum(o * p, axis=0)


def _combine(o_u, log_u, tb=512):
    return pl.pallas_call(
        _comb_body,
        grid=(HEADS, SEQ // tb),
        in_specs=[
            pl.BlockSpec((NH, tb, DH), lambda h, t: (h, t, 0)),
            pl.BlockSpec((NH, tb, 16), lambda h, t: (h, t, 0)),
        ],
        out_specs=pl.BlockSpec((1, tb, DH), lambda h, t: (h, t, 0)),
        out_shape=jax.ShapeDtypeStruct((HEADS, SEQ, DH), jnp.float32),
    )(o_u, log_u)


def _head_body(x1_ref, x2_ref, g_ref, b_ref, w_ref, bo_ref, o_ref):
    x = (x1_ref[...] + x2_ref[...]) * 0.5
    m = jnp.mean(x, axis=-1, keepdims=True)
    v = jnp.mean((x - m) ** 2, axis=-1, keepdims=True)
    x = (x - m) * jax.lax.rsqrt(v + 1e-5) * g_ref[...] + b_ref[...]
    xm = jnp.mean(x, axis=0, keepdims=True)
    o_ref[...] = jnp.dot(xm, w_ref[...], preferred_element_type=jnp.float32) + bo_ref[...]


def _final_head(x1, x2, gf, bfin, wout, bout):
    return pl.pallas_call(
        _head_body,
        in_specs=[
            pl.BlockSpec((SEQ, DIM), lambda: (0, 0)),
            pl.BlockSpec((SEQ, DIM), lambda: (0, 0)),
            pl.BlockSpec((1, DIM), lambda: (0, 0)),
            pl.BlockSpec((1, DIM), lambda: (0, 0)),
            pl.BlockSpec((DIM, OUT), lambda: (0, 0)),
            pl.BlockSpec((1, OUT), lambda: (0, 0)),
        ],
        out_specs=pl.BlockSpec((1, OUT), lambda: (0, 0)),
        out_shape=jax.ShapeDtypeStruct((1, OUT), jnp.float32),
    )(x1, x2, gf.reshape(1, DIM), bfin.reshape(1, DIM), wout,
      bout.reshape(1, OUT))


def _lsh_attention(x2, g1, b1, wqk, wv, wo, bo, x1, rotflat):
    qkv = _ln_mm(x2, jnp.concatenate([wqk, wv], axis=1),
                 jnp.zeros((2 * HEADS * DH,), jnp.float32), g1, b1)
    lbuck = _lsh_hash(qkv, rotflat)
    pos = _sortperm(lbuck)

    qk = qkv[:, :HEADS * DH].reshape(SEQ, HEADS, DH).transpose(1, 0, 2)
    v = qkv[:, HEADS * DH:].reshape(SEQ, HEADS, DH).transpose(1, 0, 2)
    rows = jnp.concatenate([qk, v], axis=-1)

    posf = pos.transpose(0, 2, 1).reshape(HEADS, S)
    rows4 = jnp.tile(rows, (1, NH, 1))
    hh = jnp.arange(HEADS)[:, None]
    sqkv = jnp.zeros((HEADS, S, 2 * DH), jnp.float32).at[hh, posf, :].set(rows4)
    tok = jnp.tile(jnp.arange(SEQ, dtype=jnp.float32)[None, :], (HEADS, NH))
    st = jnp.zeros((HEADS, S), jnp.float32).at[hh, posf].set(tok)
    stq = jnp.broadcast_to(st[:, :, None], (HEADS, S, 16))
    stl = st.reshape(HEADS, NCH, 1, BUCKET)

    so, slog = _chunk_attn(sqkv, stq, stl)

    o_u = jnp.take_along_axis(
        so, posf[:, :, None], axis=1).reshape(HEADS * NH, SEQ, DH)
    log_u = jnp.take_along_axis(
        slog, posf[:, :, None], axis=1).reshape(HEADS * NH, SEQ, 16)

    ctx = _combine(o_u, log_u).transpose(1, 0, 2).reshape(SEQ, HEADS * DH)
    return _ln_mm(ctx, wo, bo, resid=x1)


def kernel(x, params):
    p = params
    h = x[0] + p['pos_emb'][:SEQ]
    x1 = h
    x2 = h
    for i in range(DEPTH):
        rotflat = _ROT[i].reshape(DH, NH * (NB // 2))
        y1 = _lsh_attention(x2, p['g1'][i], p['b1'][i], p['Wqk'][i],
                            p['Wv'][i], p['Wo'][i], p['bo'][i], x1, rotflat)
        ff = _ln_mm(y1, p['W1'][i], p['bf1'][i], p['g2'][i], p['b2'][i],
                    act="gelu")
        y2 = _ln_mm(ff, p['W2'][i], p['bf2'][i], resid=x2)
        x1, x2 = y1, y2
    return _final_head(x1, x2, p['gf'], p['bfin'], p['Wout'], p['bout'])


# speedup vs baseline: 3.1691x; 1.7355x over previous
import functools

import jax
import jax.numpy as jnp
import numpy as np
from jax import lax
from jax.experimental import pallas as pl
from jax.experimental.pallas import tpu as pltpu
from jax.experimental.pallas import tpu_sc as plsc

B = 1
SEQ = 2048
DIM = 1024
DEPTH = 2
HEADS = 16
DH = 64
BUCKET = 64
NH = 4
OUT = 2
MAXSEQ = 2048
NB = SEQ // BUCKET
NCH = NH * NB
S = NH * SEQ
CPS = SEQ // BUCKET

_ROT = np.random.default_rng(12345).standard_normal(
    (DEPTH, DH, NH, NB // 2)).astype(np.float32)
_LT128 = np.tril(np.ones((128, 128), np.float32), -1)


def _mm_body(x_ref, w_ref, b_ref, g_ref, bb_ref, r_ref, o_ref, *, ln, act, res):
    x = x_ref[...]
    if ln:
        m = jnp.mean(x, axis=-1, keepdims=True)
        v = jnp.mean((x - m) ** 2, axis=-1, keepdims=True)
        x = (x - m) * jax.lax.rsqrt(v + 1e-5) * g_ref[...] + bb_ref[...]
    acc = jnp.dot(x, w_ref[...], preferred_element_type=jnp.float32)
    acc = acc + b_ref[...]
    if act == "gelu":
        acc = 0.5 * acc * (1.0 + jax.lax.erf(acc * (2.0 ** -0.5)))
    if res:
        acc = acc + r_ref[...]
    o_ref[...] = acc


def _ln_mm(x, w, bias, gamma=None, beta=None, resid=None, act=None,
           bm=256, bn=512):
    M, K = x.shape
    N = w.shape[1]
    ln = gamma is not None
    res = resid is not None
    if not ln:
        gamma = jnp.zeros((1, 1), jnp.float32)
        beta = jnp.zeros((1, 1), jnp.float32)
    else:
        gamma = gamma.reshape(1, K)
        beta = beta.reshape(1, K)
    if not res:
        resid = jnp.zeros((1, 1), jnp.float32)
    bias = bias.reshape(1, N)
    gk = gamma.shape[1]
    grid = (M // bm, N // bn)
    return pl.pallas_call(
        functools.partial(_mm_body, ln=ln, act=act, res=res),
        grid=grid,
        in_specs=[
            pl.BlockSpec((bm, K), lambda i, j: (i, 0)),
            pl.BlockSpec((K, bn), lambda i, j: (0, j)),
            pl.BlockSpec((1, bn), lambda i, j: (0, j)),
            pl.BlockSpec((1, gk), lambda i, j: (0, 0)),
            pl.BlockSpec((1, gk), lambda i, j: (0, 0)),
            (pl.BlockSpec((bm, bn), lambda i, j: (i, j)) if res
             else pl.BlockSpec((1, 1), lambda i, j: (0, 0))),
        ],
        out_specs=pl.BlockSpec((bm, bn), lambda i, j: (i, j)),
        out_shape=jax.ShapeDtypeStruct((M, N), jnp.float32),
    )(x, w, bias, gamma, beta, resid)


def _hash_body(qk_ref, rot_ref, o_ref):
    half = NB // 2
    for g in range(2):
        x = qk_ref[:, g * DH:(g + 1) * DH]
        r = jnp.dot(x, rot_ref[...], preferred_element_type=jnp.float32)
        for rr in range(NH):
            sl = r[:, rr * half:(rr + 1) * half]
            full = jnp.concatenate([sl, -sl], axis=1)
            mx = jnp.max(full, axis=1, keepdims=True)
            ii = lax.broadcasted_iota(jnp.int32, (SEQ, NB), 1)
            am = jnp.min(jnp.where(full == mx, ii, NB), axis=1, keepdims=True)
            o_ref[g, :, rr:rr + 1] = am.astype(jnp.int32)


def _lsh_hash(qkv, rotflat):
    return pl.pallas_call(
        _hash_body,
        grid=(HEADS // 2,),
        in_specs=[
            pl.BlockSpec((SEQ, 2 * DH), lambda h: (0, h)),
            pl.BlockSpec((DH, NH * NB // 2), lambda h: (0, 0)),
        ],
        out_specs=pl.BlockSpec((2, SEQ, NH), lambda h: (h, 0, 0)),
        out_shape=jax.ShapeDtypeStruct((HEADS, SEQ, NH), jnp.int32),
    )(qkv, rotflat)


def _sortperm_body(lb_ref, lt_ref, o_ref):
    lt = lt_ref[...]
    for r in range(NH):
        l = lb_ref[0, :, r:r + 1]
        oh = (l == lax.broadcasted_iota(jnp.int32, (SEQ, NB), 1))
        oh = oh.astype(jnp.float32)
        segs = []
        carry = jnp.zeros((1, NB), jnp.float32)
        for bb in range(SEQ // 128):
            blk = oh[bb * 128:(bb + 1) * 128, :]
            segs.append(jnp.dot(lt, blk, preferred_element_type=jnp.float32)
                        + carry)
            carry = carry + jnp.sum(blk, axis=0, keepdims=True)
        cum = jnp.concatenate(segs, axis=0)
        hist = carry
        ii = lax.broadcasted_iota(jnp.int32, (NB, NB), 0)
        jj = lax.broadcasted_iota(jnp.int32, (NB, NB), 1)
        m = (ii < jj).astype(jnp.float32)
        bstart = jnp.dot(hist, m, preferred_element_type=jnp.float32)
        pos = jnp.sum(oh * (cum + bstart), axis=1, keepdims=True)
        base = r * SEQ + pl.program_id(0) * S
        o_ref[0, :, r:r + 1] = (pos.astype(jnp.int32) + base)


def _sortperm(lbuck):
    return pl.pallas_call(
        _sortperm_body,
        grid=(HEADS,),
        in_specs=[
            pl.BlockSpec((1, SEQ, NH), lambda h: (h, 0, 0)),
            pl.BlockSpec((128, 128), lambda h: (0, 0)),
        ],
        out_specs=pl.BlockSpec((1, SEQ, NH), lambda h: (h, 0, 0)),
        out_shape=jax.ShapeDtypeStruct((HEADS, SEQ, NH), jnp.int32),
    )(lbuck, _LT128)


_SC_NC = 2
_SC_NS = 16
_SC_NW = _SC_NC * _SC_NS
_CH = 128


def _sc_mesh():
    return plsc.VectorSubcoreMesh(core_axis_name="c", subcore_axis_name="s")


def _sc_wid():
    return lax.axis_index("s") * _SC_NC + lax.axis_index("c")


@functools.cache
def _get_sc_scatter():
    @functools.partial(
        pl.kernel,
        out_type=jax.ShapeDtypeStruct((HEADS * S, 2 * DH), jnp.float32),
        mesh=_sc_mesh(),
        scratch_types=[
            pltpu.VMEM((_CH, 2 * DH), jnp.float32),
            pltpu.VMEM((SEQ // _CH, _CH), jnp.int32),
            pltpu.SemaphoreType.DMA,
        ],
    )
    def _sc_scatter(rows_hbm, gidx_hbm, out_rows, buf, idxv, sem1):
        wid = _sc_wid()
        for si in range(2):
            span = wid * 2 + si
            h = span // NH
            pltpu.sync_copy(
                gidx_hbm.at[pl.ds(span * (SEQ // _CH), SEQ // _CH)], idxv)

            def body(c, carry):
                pltpu.sync_copy(
                    rows_hbm.at[pl.ds(h * SEQ + c * _CH, _CH)], buf)
                pltpu.async_copy(buf, out_rows.at[idxv.at[c]], sem1).wait()
                return carry

            lax.fori_loop(0, SEQ // _CH, body, 0)

    return _sc_scatter


@functools.cache
def _get_sc_gather():
    @functools.partial(
        pl.kernel,
        out_type=jax.ShapeDtypeStruct((HEADS * S, 2 * DH), jnp.float32),
        mesh=_sc_mesh(),
        scratch_types=[
            pltpu.VMEM((HEADS * S // _SC_NW // _CH, _CH), jnp.int32),
            pltpu.VMEM((_CH, 2 * DH), jnp.float32),
            pltpu.SemaphoreType.DMA,
        ],
    )
    def _sc_gather(fused_hbm, gidx_hbm, o_out, idxv, obuf, sem1):
        wid = _sc_wid()
        per = HEADS * S // _SC_NW
        nch = per // _CH
        pltpu.sync_copy(gidx_hbm.at[pl.ds(wid * nch, nch)], idxv)

        def body(c, carry):
            pltpu.async_copy(fused_hbm.at[idxv.at[c]], obuf, sem1).wait()
            pltpu.sync_copy(obuf, o_out.at[pl.ds(wid * per + c * _CH, _CH)])
            return carry

        lax.fori_loop(0, nch, body, 0)

    return _sc_gather


def _attn_body(sqk_ref, ptl_ref, stq_ref, stl_ref, ptt_ref, so_ref):
    scale = np.float32(DH ** -0.5)

    def norm(k):
        n = jnp.sqrt(jnp.sum(k * k, axis=-1, keepdims=True))
        return k / jnp.maximum(n, 1e-12)

    for c in range(CPS):
        lo = c * BUCKET
        hi = lo + BUCKET
        q = sqk_ref[0, lo:hi, :DH]
        v = sqk_ref[0, lo:hi, DH:]
        qt = stq_ref[0, lo:hi, 0:1]
        ktc = stl_ref[0, c, :, :]
        if c == 0:
            kprev = ptl_ref[0, :, :DH]
            vprev = ptl_ref[0, :, DH:]
            ktp = ptt_ref[0, 0, :, :]
        else:
            kprev = sqk_ref[0, lo - BUCKET:lo, :DH]
            vprev = sqk_ref[0, lo - BUCKET:lo, DH:]
            ktp = stl_ref[0, c - 1, :, :]
        kcat = jnp.concatenate([q, kprev], axis=0)
        vcat = jnp.concatenate([v, vprev], axis=0)
        ktrow = jnp.concatenate([ktc, ktp], axis=1)
        kn = norm(kcat)
        dots = lax.dot_general(q, kn, (((1,), (1,)), ((), ())),
                               preferred_element_type=jnp.float32) * scale
        mask = qt == ktrow
        dots = jnp.where(mask, -5e4, dots)
        mx = jnp.max(dots, axis=1, keepdims=True)
        e = jnp.exp(dots - mx)
        ssum = jnp.sum(e, axis=1, keepdims=True)
        lse = mx + jnp.log(ssum)
        p = e / ssum
        o = jnp.dot(p, vcat, preferred_element_type=jnp.float32)
        row = jnp.concatenate(
            [o, jnp.broadcast_to(lse, (BUCKET, 16)),
             jnp.zeros((BUCKET, 48), jnp.float32)], axis=1)
        so_ref[0, lo:hi, :] = row


def _chunk_attn(sqkv, stq, stl):
    return pl.pallas_call(
        _attn_body,
        grid=(HEADS, NH),
        in_specs=[
            pl.BlockSpec((1, SEQ, 2 * DH), lambda h, r: (h, r, 0)),
            pl.BlockSpec((1, BUCKET, 2 * DH),
                         lambda h, r: (h, (r * CPS - 1) % NCH, 0)),
            pl.BlockSpec((1, SEQ, 16), lambda h, r: (h, r, 0)),
            pl.BlockSpec((1, CPS, 1, BUCKET), lambda h, r: (h, r, 0, 0)),
            pl.BlockSpec((1, 1, 1, BUCKET),
                         lambda h, r: (h, (r * CPS - 1) % NCH, 0, 0)),
        ],
        out_specs=pl.BlockSpec((1, SEQ, 2 * DH), lambda h, r: (h, r, 0)),
        out_shape=jax.ShapeDtypeStruct((HEADS, S, 2 * DH), jnp.float32),
    )(sqkv, sqkv, stq, stl, stl)


def _comb_body(f_ref, ctx_ref):
    f = f_ref[...]
    o = f[:, :, :DH]
    lg = f[:, :, DH:DH + 1]
    m = jnp.max(lg, axis=0, keepdims=True)
    e = jnp.exp(lg - m)
    p = e / jnp.sum(e, axis=0, keepdims=True)
    ctx_ref[0] = jnp.sum(o * p, axis=0)


def _combine(fused_u, tb=512):
    return pl.pallas_call(
        _comb_body,
        grid=(HEADS, SEQ // tb),
        in_specs=[
            pl.BlockSpec((NH, tb, 2 * DH), lambda h, t: (h, t, 0)),
        ],
        out_specs=pl.BlockSpec((1, tb, DH), lambda h, t: (h, t, 0)),
        out_shape=jax.ShapeDtypeStruct((HEADS, SEQ, DH), jnp.float32),
    )(fused_u)


def _head_body(x1_ref, x2_ref, g_ref, b_ref, w_ref, bo_ref, o_ref):
    x = (x1_ref[...] + x2_ref[...]) * 0.5
    m = jnp.mean(x, axis=-1, keepdims=True)
    v = jnp.mean((x - m) ** 2, axis=-1, keepdims=True)
    x = (x - m) * jax.lax.rsqrt(v + 1e-5) * g_ref[...] + b_ref[...]
    xm = jnp.mean(x, axis=0, keepdims=True)
    o_ref[...] = jnp.dot(xm, w_ref[...], preferred_element_type=jnp.float32) + bo_ref[...]


def _final_head(x1, x2, gf, bfin, wout, bout):
    return pl.pallas_call(
        _head_body,
        in_specs=[
            pl.BlockSpec((SEQ, DIM), lambda: (0, 0)),
            pl.BlockSpec((SEQ, DIM), lambda: (0, 0)),
            pl.BlockSpec((1, DIM), lambda: (0, 0)),
            pl.BlockSpec((1, DIM), lambda: (0, 0)),
            pl.BlockSpec((DIM, OUT), lambda: (0, 0)),
            pl.BlockSpec((1, OUT), lambda: (0, 0)),
        ],
        out_specs=pl.BlockSpec((1, OUT), lambda: (0, 0)),
        out_shape=jax.ShapeDtypeStruct((1, OUT), jnp.float32),
    )(x1, x2, gf.reshape(1, DIM), bfin.reshape(1, DIM), wout,
      bout.reshape(1, OUT))


def _lsh_attention(x2, g1, b1, wqk, wv, wo, bo, x1, rotflat):
    qkv = _ln_mm(x2, jnp.concatenate([wqk, wv], axis=1),
                 jnp.zeros((2 * HEADS * DH,), jnp.float32), g1, b1)
    lbuck = _lsh_hash(qkv, rotflat)
    pos = _sortperm(lbuck)

    qk = qkv[:, :HEADS * DH].reshape(SEQ, HEADS, DH).transpose(1, 0, 2)
    v = qkv[:, HEADS * DH:].reshape(SEQ, HEADS, DH).transpose(1, 0, 2)
    rows = jnp.concatenate([qk, v], axis=-1).reshape(HEADS * SEQ, 2 * DH)

    gidx = pos.transpose(0, 2, 1).reshape(HEADS * S // 128, 128)
    srows = _get_sc_scatter()(rows, gidx)
    sqkv = srows.reshape(HEADS, S, 2 * DH)

    posf = pos.transpose(0, 2, 1).reshape(HEADS, S) - (
        jnp.arange(HEADS, dtype=jnp.int32)[:, None] * S)
    tok = jnp.tile(jnp.arange(SEQ, dtype=jnp.float32)[None, :], (HEADS, NH))
    st2 = jnp.zeros((HEADS, S), jnp.float32).at[
        jnp.arange(HEADS)[:, None], posf].set(tok)
    stq = jnp.broadcast_to(st2[:, :, None], (HEADS, S, 16))
    stl = st2.reshape(HEADS, NCH, 1, BUCKET)

    fused = _chunk_attn(sqkv, stq, stl)

    g = _get_sc_gather()(fused.reshape(HEADS * S, 2 * DH), gidx)
    fused_u = g.reshape(HEADS * NH, SEQ, 2 * DH)

    ctx = _combine(fused_u).transpose(1, 0, 2).reshape(SEQ, HEADS * DH)
    return _ln_mm(ctx, wo, bo, resid=x1)


def kernel(x, params):
    p = params
    h = x[0] + p['pos_emb'][:SEQ]
    x1 = h
    x2 = h
    for i in range(DEPTH):
        rotflat = _ROT[i].reshape(DH, NH * (NB // 2))
        y1 = _lsh_attention(x2, p['g1'][i], p['b1'][i], p['Wqk'][i],
                            p['Wv'][i], p['Wo'][i], p['bo'][i], x1, rotflat)
        ff = _ln_mm(y1, p['W1'][i], p['bf1'][i], p['g2'][i], p['b2'][i],
                    act="gelu")
        y2 = _ln_mm(ff, p['W2'][i], p['bf2'][i], resid=x2)
        x1, x2 = y1, y2
    return _final_head(x1, x2, p['gf'], p['bfin'], p['Wout'], p['bout'])


# speedup vs baseline: 3.1793x; 1.0032x over previous
import functools

import jax
import jax.numpy as jnp
import numpy as np
from jax import lax
from jax.experimental import pallas as pl
from jax.experimental.pallas import tpu as pltpu
from jax.experimental.pallas import tpu_sc as plsc

B = 1
SEQ = 2048
DIM = 1024
DEPTH = 2
HEADS = 16
DH = 64
BUCKET = 64
NH = 4
OUT = 2
MAXSEQ = 2048
NB = SEQ // BUCKET
NCH = NH * NB
S = NH * SEQ
CPS = SEQ // BUCKET

_ROT = np.random.default_rng(12345).standard_normal(
    (DEPTH, DH, NH, NB // 2)).astype(np.float32)
_LT128 = np.tril(np.ones((128, 128), np.float32), -1)


def _mm_body(x_ref, w_ref, b_ref, g_ref, bb_ref, r_ref, o_ref, *, ln, act, res):
    x = x_ref[...]
    if ln:
        m = jnp.mean(x, axis=-1, keepdims=True)
        v = jnp.mean((x - m) ** 2, axis=-1, keepdims=True)
        x = (x - m) * jax.lax.rsqrt(v + 1e-5) * g_ref[...] + bb_ref[...]
    acc = jnp.dot(x, w_ref[...], preferred_element_type=jnp.float32)
    acc = acc + b_ref[...]
    if act == "gelu":
        acc = 0.5 * acc * (1.0 + jax.lax.erf(acc * (2.0 ** -0.5)))
    if res:
        acc = acc + r_ref[...]
    o_ref[...] = acc


def _ln_mm(x, w, bias, gamma=None, beta=None, resid=None, act=None,
           bm=256, bn=512):
    M, K = x.shape
    N = w.shape[1]
    ln = gamma is not None
    res = resid is not None
    if not ln:
        gamma = jnp.zeros((1, 1), jnp.float32)
        beta = jnp.zeros((1, 1), jnp.float32)
    else:
        gamma = gamma.reshape(1, K)
        beta = beta.reshape(1, K)
    if not res:
        resid = jnp.zeros((1, 1), jnp.float32)
    bias = bias.reshape(1, N)
    gk = gamma.shape[1]
    grid = (M // bm, N // bn)
    return pl.pallas_call(
        functools.partial(_mm_body, ln=ln, act=act, res=res),
        grid=grid,
        in_specs=[
            pl.BlockSpec((bm, K), lambda i, j: (i, 0)),
            pl.BlockSpec((K, bn), lambda i, j: (0, j)),
            pl.BlockSpec((1, bn), lambda i, j: (0, j)),
            pl.BlockSpec((1, gk), lambda i, j: (0, 0)),
            pl.BlockSpec((1, gk), lambda i, j: (0, 0)),
            (pl.BlockSpec((bm, bn), lambda i, j: (i, j)) if res
             else pl.BlockSpec((1, 1), lambda i, j: (0, 0))),
        ],
        out_specs=pl.BlockSpec((bm, bn), lambda i, j: (i, j)),
        out_shape=jax.ShapeDtypeStruct((M, N), jnp.float32),
    )(x, w, bias, gamma, beta, resid)


def _hash_body(qk_ref, rot_ref, o_ref):
    half = NB // 2
    for g in range(2):
        x = qk_ref[:, g * DH:(g + 1) * DH]
        r = jnp.dot(x, rot_ref[...], preferred_element_type=jnp.float32)
        for rr in range(NH):
            sl = r[:, rr * half:(rr + 1) * half]
            full = jnp.concatenate([sl, -sl], axis=1)
            mx = jnp.max(full, axis=1, keepdims=True)
            ii = lax.broadcasted_iota(jnp.int32, (SEQ, NB), 1)
            am = jnp.min(jnp.where(full == mx, ii, NB), axis=1, keepdims=True)
            o_ref[g, :, rr:rr + 1] = am.astype(jnp.int32)


def _lsh_hash(qkv, rotflat):
    return pl.pallas_call(
        _hash_body,
        grid=(HEADS // 2,),
        in_specs=[
            pl.BlockSpec((SEQ, 2 * DH), lambda h: (0, h)),
            pl.BlockSpec((DH, NH * NB // 2), lambda h: (0, 0)),
        ],
        out_specs=pl.BlockSpec((2, SEQ, NH), lambda h: (h, 0, 0)),
        out_shape=jax.ShapeDtypeStruct((HEADS, SEQ, NH), jnp.int32),
    )(qkv, rotflat)


def _sortperm_body(lb_ref, lt_ref, o_ref):
    lt = lt_ref[...]
    for r in range(NH):
        l = lb_ref[0, :, r:r + 1]
        oh = (l == lax.broadcasted_iota(jnp.int32, (SEQ, NB), 1))
        oh = oh.astype(jnp.float32)
        segs = []
        carry = jnp.zeros((1, NB), jnp.float32)
        for bb in range(SEQ // 128):
            blk = oh[bb * 128:(bb + 1) * 128, :]
            segs.append(jnp.dot(lt, blk, preferred_element_type=jnp.float32)
                        + carry)
            carry = carry + jnp.sum(blk, axis=0, keepdims=True)
        cum = jnp.concatenate(segs, axis=0)
        hist = carry
        ii = lax.broadcasted_iota(jnp.int32, (NB, NB), 0)
        jj = lax.broadcasted_iota(jnp.int32, (NB, NB), 1)
        m = (ii < jj).astype(jnp.float32)
        bstart = jnp.dot(hist, m, preferred_element_type=jnp.float32)
        pos = jnp.sum(oh * (cum + bstart), axis=1, keepdims=True)
        base = r * SEQ + pl.program_id(0) * S
        o_ref[0, :, r:r + 1] = (pos.astype(jnp.int32) + base)


def _sortperm(lbuck):
    return pl.pallas_call(
        _sortperm_body,
        grid=(HEADS,),
        in_specs=[
            pl.BlockSpec((1, SEQ, NH), lambda h: (h, 0, 0)),
            pl.BlockSpec((128, 128), lambda h: (0, 0)),
        ],
        out_specs=pl.BlockSpec((1, SEQ, NH), lambda h: (h, 0, 0)),
        out_shape=jax.ShapeDtypeStruct((HEADS, SEQ, NH), jnp.int32),
    )(lbuck, _LT128)


_SC_NC = 2
_SC_NS = 16
_SC_NW = _SC_NC * _SC_NS
_CH = 128


def _sc_mesh():
    return plsc.VectorSubcoreMesh(core_axis_name="c", subcore_axis_name="s")


def _sc_wid():
    return lax.axis_index("s") * _SC_NC + lax.axis_index("c")


@functools.cache
def _get_sc_scatter():
    @functools.partial(
        pl.kernel,
        out_type=jax.ShapeDtypeStruct((HEADS * S, 2 * DH), jnp.float32),
        mesh=_sc_mesh(),
        scratch_types=[
            pltpu.VMEM((2, _CH, 2 * DH), jnp.float32),
            pltpu.VMEM((SEQ // _CH, _CH), jnp.int32),
            pltpu.SemaphoreType.DMA,
            pltpu.SemaphoreType.DMA,
        ],
    )
    def _sc_scatter(rows_hbm, gidx_hbm, out_rows, buf, idxv, lsem, ssem):
        wid = _sc_wid()
        nck = SEQ // _CH
        for si in range(2):
            span = wid * 2 + si
            h = span // NH
            pltpu.sync_copy(
                gidx_hbm.at[pl.ds(span * (SEQ // _CH), SEQ // _CH)], idxv)
            pltpu.sync_copy(rows_hbm.at[pl.ds(h * SEQ, _CH)], buf.at[0])
            sc_prev = pltpu.async_copy(buf.at[0], out_rows.at[idxv.at[0]],
                                       ssem)
            for c in range(1, nck):
                b = c % 2
                ld = pltpu.async_copy(
                    rows_hbm.at[pl.ds(h * SEQ + c * _CH, _CH)], buf.at[b],
                    lsem)
                sc_prev.wait()
                ld.wait()
                sc_prev = pltpu.async_copy(
                    buf.at[b], out_rows.at[idxv.at[c]], ssem)
            sc_prev.wait()

    return _sc_scatter


@functools.cache
def _get_sc_gather():
    @functools.partial(
        pl.kernel,
        out_type=jax.ShapeDtypeStruct((HEADS * S, 2 * DH), jnp.float32),
        mesh=_sc_mesh(),
        scratch_types=[
            pltpu.VMEM((HEADS * S // _SC_NW // _CH, _CH), jnp.int32),
            pltpu.VMEM((2, _CH, 2 * DH), jnp.float32),
            pltpu.SemaphoreType.DMA,
            pltpu.SemaphoreType.DMA,
        ],
    )
    def _sc_gather(fused_hbm, gidx_hbm, o_out, idxv, obuf, gsem, wsem):
        wid = _sc_wid()
        per = HEADS * S // _SC_NW
        nch = per // _CH
        pltpu.sync_copy(gidx_hbm.at[pl.ds(wid * nch, nch)], idxv)

        g_prev = pltpu.async_copy(fused_hbm.at[idxv.at[0]], obuf.at[0], gsem)
        w_prev = None
        for c in range(1, nch + 1):
            g_prev.wait()
            if w_prev is not None:
                w_prev.wait()
            w_prev = pltpu.async_copy(
                obuf.at[(c - 1) % 2],
                o_out.at[pl.ds(wid * per + (c - 1) * _CH, _CH)], wsem)
            if c < nch:
                g_prev = pltpu.async_copy(
                    fused_hbm.at[idxv.at[c]], obuf.at[c % 2], gsem)
        w_prev.wait()

    return _sc_gather


def _attn_body(sqk_ref, ptl_ref, stq_ref, stl_ref, ptt_ref, so_ref):
    scale = np.float32(DH ** -0.5)

    def norm(k):
        n = jnp.sqrt(jnp.sum(k * k, axis=-1, keepdims=True))
        return k / jnp.maximum(n, 1e-12)

    for c in range(CPS):
        lo = c * BUCKET
        hi = lo + BUCKET
        q = sqk_ref[0, lo:hi, :DH]
        v = sqk_ref[0, lo:hi, DH:]
        qt = stq_ref[0, lo:hi, 0:1]
        ktc = stl_ref[0, c, :, :]
        if c == 0:
            kprev = ptl_ref[0, :, :DH]
            vprev = ptl_ref[0, :, DH:]
            ktp = ptt_ref[0, 0, :, :]
        else:
            kprev = sqk_ref[0, lo - BUCKET:lo, :DH]
            vprev = sqk_ref[0, lo - BUCKET:lo, DH:]
            ktp = stl_ref[0, c - 1, :, :]
        kcat = jnp.concatenate([q, kprev], axis=0)
        vcat = jnp.concatenate([v, vprev], axis=0)
        ktrow = jnp.concatenate([ktc, ktp], axis=1)
        kn = norm(kcat)
        dots = lax.dot_general(q, kn, (((1,), (1,)), ((), ())),
                               preferred_element_type=jnp.float32) * scale
        mask = qt == ktrow
        dots = jnp.where(mask, -5e4, dots)
        mx = jnp.max(dots, axis=1, keepdims=True)
        e = jnp.exp(dots - mx)
        ssum = jnp.sum(e, axis=1, keepdims=True)
        lse = mx + jnp.log(ssum)
        p = e / ssum
        o = jnp.dot(p, vcat, preferred_element_type=jnp.float32)
        row = jnp.concatenate(
            [o, jnp.broadcast_to(lse, (BUCKET, 16)),
             jnp.zeros((BUCKET, 48), jnp.float32)], axis=1)
        so_ref[0, lo:hi, :] = row


def _chunk_attn(sqkv, stq, stl):
    return pl.pallas_call(
        _attn_body,
        grid=(HEADS, NH),
        in_specs=[
            pl.BlockSpec((1, SEQ, 2 * DH), lambda h, r: (h, r, 0)),
            pl.BlockSpec((1, BUCKET, 2 * DH),
                         lambda h, r: (h, (r * CPS - 1) % NCH, 0)),
            pl.BlockSpec((1, SEQ, 16), lambda h, r: (h, r, 0)),
            pl.BlockSpec((1, CPS, 1, BUCKET), lambda h, r: (h, r, 0, 0)),
            pl.BlockSpec((1, 1, 1, BUCKET),
                         lambda h, r: (h, (r * CPS - 1) % NCH, 0, 0)),
        ],
        out_specs=pl.BlockSpec((1, SEQ, 2 * DH), lambda h, r: (h, r, 0)),
        out_shape=jax.ShapeDtypeStruct((HEADS, S, 2 * DH), jnp.float32),
    )(sqkv, sqkv, stq, stl, stl)


def _comb_body(f_ref, ctx_ref):
    f = f_ref[...]
    o = f[:, :, :DH]
    lg = f[:, :, DH:DH + 1]
    m = jnp.max(lg, axis=0, keepdims=True)
    e = jnp.exp(lg - m)
    p = e / jnp.sum(e, axis=0, keepdims=True)
    ctx_ref[0] = jnp.sum(o * p, axis=0)


def _combine(fused_u, tb=512):
    return pl.pallas_call(
        _comb_body,
        grid=(HEADS, SEQ // tb),
        in_specs=[
            pl.BlockSpec((NH, tb, 2 * DH), lambda h, t: (h, t, 0)),
        ],
        out_specs=pl.BlockSpec((1, tb, DH), lambda h, t: (h, t, 0)),
        out_shape=jax.ShapeDtypeStruct((HEADS, SEQ, DH), jnp.float32),
    )(fused_u)


def _head_body(x1_ref, x2_ref, g_ref, b_ref, w_ref, bo_ref, o_ref):
    x = (x1_ref[...] + x2_ref[...]) * 0.5
    m = jnp.mean(x, axis=-1, keepdims=True)
    v = jnp.mean((x - m) ** 2, axis=-1, keepdims=True)
    x = (x - m) * jax.lax.rsqrt(v + 1e-5) * g_ref[...] + b_ref[...]
    xm = jnp.mean(x, axis=0, keepdims=True)
    o_ref[...] = jnp.dot(xm, w_ref[...], preferred_element_type=jnp.float32) + bo_ref[...]


def _final_head(x1, x2, gf, bfin, wout, bout):
    return pl.pallas_call(
        _head_body,
        in_specs=[
            pl.BlockSpec((SEQ, DIM), lambda: (0, 0)),
            pl.BlockSpec((SEQ, DIM), lambda: (0, 0)),
            pl.BlockSpec((1, DIM), lambda: (0, 0)),
            pl.BlockSpec((1, DIM), lambda: (0, 0)),
            pl.BlockSpec((DIM, OUT), lambda: (0, 0)),
            pl.BlockSpec((1, OUT), lambda: (0, 0)),
        ],
        out_specs=pl.BlockSpec((1, OUT), lambda: (0, 0)),
        out_shape=jax.ShapeDtypeStruct((1, OUT), jnp.float32),
    )(x1, x2, gf.reshape(1, DIM), bfin.reshape(1, DIM), wout,
      bout.reshape(1, OUT))


def _lsh_attention(x2, g1, b1, wqk, wv, wo, bo, x1, rotflat):
    qkv = _ln_mm(x2, jnp.concatenate([wqk, wv], axis=1),
                 jnp.zeros((2 * HEADS * DH,), jnp.float32), g1, b1)
    lbuck = _lsh_hash(qkv, rotflat)
    pos = _sortperm(lbuck)

    qk = qkv[:, :HEADS * DH].reshape(SEQ, HEADS, DH).transpose(1, 0, 2)
    v = qkv[:, HEADS * DH:].reshape(SEQ, HEADS, DH).transpose(1, 0, 2)
    rows = jnp.concatenate([qk, v], axis=-1).reshape(HEADS * SEQ, 2 * DH)

    gidx = pos.transpose(0, 2, 1).reshape(HEADS * S // 128, 128)
    srows = _get_sc_scatter()(rows, gidx)
    sqkv = srows.reshape(HEADS, S, 2 * DH)

    posf = pos.transpose(0, 2, 1).reshape(HEADS, S) - (
        jnp.arange(HEADS, dtype=jnp.int32)[:, None] * S)
    tok = jnp.tile(jnp.arange(SEQ, dtype=jnp.float32)[None, :], (HEADS, NH))
    st2 = jnp.zeros((HEADS, S), jnp.float32).at[
        jnp.arange(HEADS)[:, None], posf].set(tok)
    stq = jnp.broadcast_to(st2[:, :, None], (HEADS, S, 16))
    stl = st2.reshape(HEADS, NCH, 1, BUCKET)

    fused = _chunk_attn(sqkv, stq, stl)

    g = _get_sc_gather()(fused.reshape(HEADS * S, 2 * DH), gidx)
    fused_u = g.reshape(HEADS * NH, SEQ, 2 * DH)

    ctx = _combine(fused_u).transpose(1, 0, 2).reshape(SEQ, HEADS * DH)
    return _ln_mm(ctx, wo, bo, resid=x1)


def kernel(x, params):
    p = params
    h = x[0] + p['pos_emb'][:SEQ]
    x1 = h
    x2 = h
    for i in range(DEPTH):
        rotflat = _ROT[i].reshape(DH, NH * (NB // 2))
        y1 = _lsh_attention(x2, p['g1'][i], p['b1'][i], p['Wqk'][i],
                            p['Wv'][i], p['Wo'][i], p['bo'][i], x1, rotflat)
        ff = _ln_mm(y1, p['W1'][i], p['bf1'][i], p['g2'][i], p['b2'][i],
                    act="gelu")
        y2 = _ln_mm(ff, p['W2'][i], p['bf2'][i], resid=x2)
        x1, x2 = y1, y2
    return _final_head(x1, x2, p['gf'], p['bfin'], p['Wout'], p['bout'])


# speedup vs baseline: 3.5026x; 1.1017x over previous
import functools

import jax
import jax.numpy as jnp
import numpy as np
from jax import lax
from jax.experimental import pallas as pl
from jax.experimental.pallas import tpu as pltpu
from jax.experimental.pallas import tpu_sc as plsc

B = 1
SEQ = 2048
DIM = 1024
DEPTH = 2
HEADS = 16
DH = 64
BUCKET = 64
NH = 4
OUT = 2
MAXSEQ = 2048
NB = SEQ // BUCKET
NCH = NH * NB
S = NH * SEQ
CPS = SEQ // BUCKET

_ROT = np.random.default_rng(12345).standard_normal(
    (DEPTH, DH, NH, NB // 2)).astype(np.float32)
_LT128 = np.tril(np.ones((128, 128), np.float32), -1)


def _mm_body(x_ref, w_ref, b_ref, g_ref, bb_ref, r_ref, o_ref, *, ln, act, res):
    x = x_ref[...]
    if ln:
        m = jnp.mean(x, axis=-1, keepdims=True)
        v = jnp.mean((x - m) ** 2, axis=-1, keepdims=True)
        x = (x - m) * jax.lax.rsqrt(v + 1e-5) * g_ref[...] + bb_ref[...]
    acc = jnp.dot(x, w_ref[...], preferred_element_type=jnp.float32)
    acc = acc + b_ref[...]
    if act == "gelu":
        acc = 0.5 * acc * (1.0 + jax.lax.erf(acc * (2.0 ** -0.5)))
    if res:
        acc = acc + r_ref[...]
    o_ref[...] = acc


def _ln_mm(x, w, bias, gamma=None, beta=None, resid=None, act=None,
           bm=256, bn=512):
    M, K = x.shape
    N = w.shape[1]
    ln = gamma is not None
    res = resid is not None
    if not ln:
        gamma = jnp.zeros((1, 1), jnp.float32)
        beta = jnp.zeros((1, 1), jnp.float32)
    else:
        gamma = gamma.reshape(1, K)
        beta = beta.reshape(1, K)
    if not res:
        resid = jnp.zeros((1, 1), jnp.float32)
    bias = bias.reshape(1, N)
    gk = gamma.shape[1]
    grid = (M // bm, N // bn)
    return pl.pallas_call(
        functools.partial(_mm_body, ln=ln, act=act, res=res),
        grid=grid,
        in_specs=[
            pl.BlockSpec((bm, K), lambda i, j: (i, 0)),
            pl.BlockSpec((K, bn), lambda i, j: (0, j)),
            pl.BlockSpec((1, bn), lambda i, j: (0, j)),
            pl.BlockSpec((1, gk), lambda i, j: (0, 0)),
            pl.BlockSpec((1, gk), lambda i, j: (0, 0)),
            (pl.BlockSpec((bm, bn), lambda i, j: (i, j)) if res
             else pl.BlockSpec((1, 1), lambda i, j: (0, 0))),
        ],
        out_specs=pl.BlockSpec((bm, bn), lambda i, j: (i, j)),
        out_shape=jax.ShapeDtypeStruct((M, N), jnp.float32),
    )(x, w, bias, gamma, beta, resid)


def _hash_body(qk_ref, rot_ref, o_ref):
    half = NB // 2
    for g in range(2):
        x = qk_ref[:, g * DH:(g + 1) * DH]
        r = jnp.dot(x, rot_ref[...], preferred_element_type=jnp.float32)
        for rr in range(NH):
            sl = r[:, rr * half:(rr + 1) * half]
            full = jnp.concatenate([sl, -sl], axis=1)
            mx = jnp.max(full, axis=1, keepdims=True)
            ii = lax.broadcasted_iota(jnp.int32, (SEQ, NB), 1)
            am = jnp.min(jnp.where(full == mx, ii, NB), axis=1, keepdims=True)
            o_ref[g, :, rr:rr + 1] = am.astype(jnp.int32)


def _lsh_hash(qkv, rotflat):
    return pl.pallas_call(
        _hash_body,
        grid=(HEADS // 2,),
        in_specs=[
            pl.BlockSpec((SEQ, 2 * DH), lambda h: (0, h)),
            pl.BlockSpec((DH, NH * NB // 2), lambda h: (0, 0)),
        ],
        out_specs=pl.BlockSpec((2, SEQ, NH), lambda h: (h, 0, 0)),
        out_shape=jax.ShapeDtypeStruct((HEADS, SEQ, NH), jnp.int32),
    )(qkv, rotflat)


def _sortperm_body(lb_ref, lt_ref, o_ref):
    lt = lt_ref[...]
    for r in range(NH):
        l = lb_ref[0, :, r:r + 1]
        oh = (l == lax.broadcasted_iota(jnp.int32, (SEQ, NB), 1))
        oh = oh.astype(jnp.float32)
        segs = []
        carry = jnp.zeros((1, NB), jnp.float32)
        for bb in range(SEQ // 128):
            blk = oh[bb * 128:(bb + 1) * 128, :]
            segs.append(jnp.dot(lt, blk, preferred_element_type=jnp.float32)
                        + carry)
            carry = carry + jnp.sum(blk, axis=0, keepdims=True)
        cum = jnp.concatenate(segs, axis=0)
        hist = carry
        ii = lax.broadcasted_iota(jnp.int32, (NB, NB), 0)
        jj = lax.broadcasted_iota(jnp.int32, (NB, NB), 1)
        m = (ii < jj).astype(jnp.float32)
        bstart = jnp.dot(hist, m, preferred_element_type=jnp.float32)
        pos = jnp.sum(oh * (cum + bstart), axis=1, keepdims=True)
        base = r * SEQ + pl.program_id(0) * S
        o_ref[0, :, r:r + 1] = (pos.astype(jnp.int32) + base)


def _sortperm(lbuck):
    return pl.pallas_call(
        _sortperm_body,
        grid=(HEADS,),
        in_specs=[
            pl.BlockSpec((1, SEQ, NH), lambda h: (h, 0, 0)),
            pl.BlockSpec((128, 128), lambda h: (0, 0)),
        ],
        out_specs=pl.BlockSpec((1, SEQ, NH), lambda h: (h, 0, 0)),
        out_shape=jax.ShapeDtypeStruct((HEADS, SEQ, NH), jnp.int32),
    )(lbuck, _LT128)


_SC_NC = 2
_SC_NS = 16
_SC_NW = _SC_NC * _SC_NS
_CH = 128


def _sc_mesh():
    return plsc.VectorSubcoreMesh(core_axis_name="c", subcore_axis_name="s")


def _sc_wid():
    return lax.axis_index("s") * _SC_NC + lax.axis_index("c")


@functools.cache
def _get_sc_scatter():
    @functools.partial(
        pl.kernel,
        out_type=jax.ShapeDtypeStruct((HEADS * S, 2 * DH), jnp.float32),
        mesh=_sc_mesh(),
        scratch_types=[
            pltpu.VMEM((2, _CH, 2 * DH), jnp.float32),
            pltpu.VMEM((SEQ // _CH, _CH), jnp.int32),
            pltpu.SemaphoreType.DMA,
            pltpu.SemaphoreType.DMA,
        ],
    )
    def _sc_scatter(rows_hbm, gidx_hbm, out_rows, buf, idxv, lsem, ssem):
        wid = _sc_wid()
        nck = SEQ // _CH
        for si in range(2):
            span = wid * 2 + si
            h = span // NH
            pltpu.sync_copy(
                gidx_hbm.at[pl.ds(span * (SEQ // _CH), SEQ // _CH)], idxv)
            pltpu.sync_copy(rows_hbm.at[pl.ds(h * SEQ, _CH)], buf.at[0])
            sc_prev = pltpu.async_copy(buf.at[0], out_rows.at[idxv.at[0]],
                                       ssem)
            for c in range(1, nck):
                b = c % 2
                ld = pltpu.async_copy(
                    rows_hbm.at[pl.ds(h * SEQ + c * _CH, _CH)], buf.at[b],
                    lsem)
                sc_prev.wait()
                ld.wait()
                sc_prev = pltpu.async_copy(
                    buf.at[b], out_rows.at[idxv.at[c]], ssem)
            sc_prev.wait()

    return _sc_scatter


@functools.cache
def _get_sc_gather():
    @functools.partial(
        pl.kernel,
        out_type=jax.ShapeDtypeStruct((HEADS * S, 2 * DH), jnp.float32),
        mesh=_sc_mesh(),
        scratch_types=[
            pltpu.VMEM((HEADS * S // _SC_NW // _CH, _CH), jnp.int32),
            pltpu.VMEM((2, _CH, 2 * DH), jnp.float32),
            pltpu.SemaphoreType.DMA,
            pltpu.SemaphoreType.DMA,
        ],
    )
    def _sc_gather(fused_hbm, gidx_hbm, o_out, idxv, obuf, gsem, wsem):
        wid = _sc_wid()
        per = HEADS * S // _SC_NW
        nch = per // _CH
        pltpu.sync_copy(gidx_hbm.at[pl.ds(wid * nch, nch)], idxv)

        g_prev = pltpu.async_copy(fused_hbm.at[idxv.at[0]], obuf.at[0], gsem)
        w_prev = None
        for c in range(1, nch + 1):
            g_prev.wait()
            if w_prev is not None:
                w_prev.wait()
            w_prev = pltpu.async_copy(
                obuf.at[(c - 1) % 2],
                o_out.at[pl.ds(wid * per + (c - 1) * _CH, _CH)], wsem)
            if c < nch:
                g_prev = pltpu.async_copy(
                    fused_hbm.at[idxv.at[c]], obuf.at[c % 2], gsem)
        w_prev.wait()

    return _sc_gather


_GRP = 4


def _attn_body(sqk_ref, ptl_ref, stq_ref, stl_ref, ptt_ref, so_ref):
    scale = np.float32(DH ** -0.5)
    GB = _GRP * BUCKET
    WB = (_GRP + 1) * BUCKET

    ri = lax.broadcasted_iota(jnp.int32, (GB, WB), 0) // BUCKET
    ci = lax.broadcasted_iota(jnp.int32, (GB, WB), 1) // BUCKET
    dd = ci - ri
    invalid = (dd < 0) | (dd > 1)

    for g in range(CPS // _GRP):
        lo = g * GB
        q = sqk_ref[0, lo:lo + GB, :DH]
        qt = stq_ref[0, lo:lo + GB, 0:1]
        if g == 0:
            kv = jnp.concatenate(
                [ptl_ref[0, :, :], sqk_ref[0, 0:GB, :]], axis=0)
            ktw = jnp.concatenate(
                [ptt_ref[0, 0, :, :]] +
                [stl_ref[0, j, :, :] for j in range(_GRP)], axis=1)
        else:
            kv = sqk_ref[0, lo - BUCKET:lo + GB, :]
            ktw = jnp.concatenate(
                [stl_ref[0, g * _GRP - 1 + j, :, :] for j in range(_GRP + 1)],
                axis=1)
        k = kv[:, :DH]
        vv = kv[:, DH:]
        n = jnp.sqrt(jnp.sum(k * k, axis=-1, keepdims=True))
        kn = k / jnp.maximum(n, 1e-12)
        dots = lax.dot_general(q, kn, (((1,), (1,)), ((), ())),
                               preferred_element_type=jnp.float32) * scale
        dots = jnp.where(qt == ktw, -5e4, dots)
        dots = jnp.where(invalid, -1e9, dots)
        mx = jnp.max(dots, axis=1, keepdims=True)
        e = jnp.exp(dots - mx)
        ssum = jnp.sum(e, axis=1, keepdims=True)
        lse = mx + jnp.log(ssum)
        p = e / ssum
        o = jnp.dot(p, vv, preferred_element_type=jnp.float32)
        row = jnp.concatenate(
            [o, jnp.broadcast_to(lse, (GB, 16)),
             jnp.zeros((GB, 48), jnp.float32)], axis=1)
        so_ref[0, lo:lo + GB, :] = row


def _chunk_attn(sqkv, stq, stl):
    return pl.pallas_call(
        _attn_body,
        grid=(HEADS, NH),
        in_specs=[
            pl.BlockSpec((1, SEQ, 2 * DH), lambda h, r: (h, r, 0)),
            pl.BlockSpec((1, BUCKET, 2 * DH),
                         lambda h, r: (h, (r * CPS - 1) % NCH, 0)),
            pl.BlockSpec((1, SEQ, 16), lambda h, r: (h, r, 0)),
            pl.BlockSpec((1, CPS, 1, BUCKET), lambda h, r: (h, r, 0, 0)),
            pl.BlockSpec((1, 1, 1, BUCKET),
                         lambda h, r: (h, (r * CPS - 1) % NCH, 0, 0)),
        ],
        out_specs=pl.BlockSpec((1, SEQ, 2 * DH), lambda h, r: (h, r, 0)),
        out_shape=jax.ShapeDtypeStruct((HEADS, S, 2 * DH), jnp.float32),
    )(sqkv, sqkv, stq, stl, stl)


def _comb_body(f_ref, ctx_ref):
    f = f_ref[...]
    o = f[:, :, :DH]
    lg = f[:, :, DH:DH + 1]
    m = jnp.max(lg, axis=0, keepdims=True)
    e = jnp.exp(lg - m)
    p = e / jnp.sum(e, axis=0, keepdims=True)
    ctx_ref[0] = jnp.sum(o * p, axis=0)


def _combine(fused_u, tb=512):
    return pl.pallas_call(
        _comb_body,
        grid=(HEADS, SEQ // tb),
        in_specs=[
            pl.BlockSpec((NH, tb, 2 * DH), lambda h, t: (h, t, 0)),
        ],
        out_specs=pl.BlockSpec((1, tb, DH), lambda h, t: (h, t, 0)),
        out_shape=jax.ShapeDtypeStruct((HEADS, SEQ, DH), jnp.float32),
    )(fused_u)


def _head_body(x1_ref, x2_ref, g_ref, b_ref, w_ref, bo_ref, o_ref):
    x = (x1_ref[...] + x2_ref[...]) * 0.5
    m = jnp.mean(x, axis=-1, keepdims=True)
    v = jnp.mean((x - m) ** 2, axis=-1, keepdims=True)
    x = (x - m) * jax.lax.rsqrt(v + 1e-5) * g_ref[...] + b_ref[...]
    xm = jnp.mean(x, axis=0, keepdims=True)
    o_ref[...] = jnp.dot(xm, w_ref[...], preferred_element_type=jnp.float32) + bo_ref[...]


def _final_head(x1, x2, gf, bfin, wout, bout):
    return pl.pallas_call(
        _head_body,
        in_specs=[
            pl.BlockSpec((SEQ, DIM), lambda: (0, 0)),
            pl.BlockSpec((SEQ, DIM), lambda: (0, 0)),
            pl.BlockSpec((1, DIM), lambda: (0, 0)),
            pl.BlockSpec((1, DIM), lambda: (0, 0)),
            pl.BlockSpec((DIM, OUT), lambda: (0, 0)),
            pl.BlockSpec((1, OUT), lambda: (0, 0)),
        ],
        out_specs=pl.BlockSpec((1, OUT), lambda: (0, 0)),
        out_shape=jax.ShapeDtypeStruct((1, OUT), jnp.float32),
    )(x1, x2, gf.reshape(1, DIM), bfin.reshape(1, DIM), wout,
      bout.reshape(1, OUT))


def _lsh_attention(x2, g1, b1, wqk, wv, wo, bo, x1, rotflat):
    qkv = _ln_mm(x2, jnp.concatenate([wqk, wv], axis=1),
                 jnp.zeros((2 * HEADS * DH,), jnp.float32), g1, b1)
    lbuck = _lsh_hash(qkv, rotflat)
    pos = _sortperm(lbuck)

    qk = qkv[:, :HEADS * DH].reshape(SEQ, HEADS, DH).transpose(1, 0, 2)
    v = qkv[:, HEADS * DH:].reshape(SEQ, HEADS, DH).transpose(1, 0, 2)
    rows = jnp.concatenate([qk, v], axis=-1).reshape(HEADS * SEQ, 2 * DH)

    gidx = pos.transpose(0, 2, 1).reshape(HEADS * S // 128, 128)
    srows = _get_sc_scatter()(rows, gidx)
    sqkv = srows.reshape(HEADS, S, 2 * DH)

    posf = pos.transpose(0, 2, 1).reshape(HEADS, S) - (
        jnp.arange(HEADS, dtype=jnp.int32)[:, None] * S)
    tok = jnp.tile(jnp.arange(SEQ, dtype=jnp.float32)[None, :], (HEADS, NH))
    st2 = jnp.zeros((HEADS, S), jnp.float32).at[
        jnp.arange(HEADS)[:, None], posf].set(tok)
    stq = jnp.broadcast_to(st2[:, :, None], (HEADS, S, 16))
    stl = st2.reshape(HEADS, NCH, 1, BUCKET)

    fused = _chunk_attn(sqkv, stq, stl)

    g = _get_sc_gather()(fused.reshape(HEADS * S, 2 * DH), gidx)
    fused_u = g.reshape(HEADS * NH, SEQ, 2 * DH)

    ctx = _combine(fused_u).transpose(1, 0, 2).reshape(SEQ, HEADS * DH)
    return _ln_mm(ctx, wo, bo, resid=x1)


def kernel(x, params):
    p = params
    h = x[0] + p['pos_emb'][:SEQ]
    x1 = h
    x2 = h
    for i in range(DEPTH):
        rotflat = _ROT[i].reshape(DH, NH * (NB // 2))
        y1 = _lsh_attention(x2, p['g1'][i], p['b1'][i], p['Wqk'][i],
                            p['Wv'][i], p['Wo'][i], p['bo'][i], x1, rotflat)
        ff = _ln_mm(y1, p['W1'][i], p['bf1'][i], p['g2'][i], p['b2'][i],
                    act="gelu")
        y2 = _ln_mm(ff, p['W2'][i], p['bf2'][i], resid=x2)
        x1, x2 = y1, y2
    return _final_head(x1, x2, p['gf'], p['bfin'], p['Wout'], p['bout'])
